# speedup vs baseline: 1.3119x; 1.3119x over previous
import functools

import jax
import jax.numpy as jnp
from jax import lax
from jax.experimental import pallas as pl
from jax.experimental.pallas import tpu as pltpu
from jax.experimental.pallas import tpu_sc as plsc

N_VOCAB = 50257
N_CTX = 2048
N_EMBED = 768
BATCH = 4

L = 16
NC, NS = 2, 16
NW = NC * NS
B = BATCH * N_CTX
BPW = B // NW
CHUNK = 64
NCHUNK = BPW // CHUNK
VPR = N_EMBED // L

_mesh = plsc.VectorSubcoreMesh(core_axis_name="c", subcore_axis_name="s")


@functools.partial(
    pl.kernel,
    mesh=_mesh,
    out_type=jax.ShapeDtypeStruct((B, N_EMBED), jnp.float32),
    scratch_types=[
        pltpu.VMEM((BPW,), jnp.int32),
        pltpu.VMEM((CHUNK, N_EMBED), jnp.float32),
        pltpu.VMEM((CHUNK, N_EMBED), jnp.float32),
        pltpu.SemaphoreType.DMA,
    ],
)
def _embed_lookup(x_hbm, wte_hbm, wpe_hbm, out_hbm, idx_v, tok_v, pos_v, sem):
    wid = lax.axis_index("s") * NC + lax.axis_index("c")
    base = wid * BPW
    t_base = lax.rem(base, N_CTX)

    pltpu.sync_copy(x_hbm.at[pl.ds(base, BPW)], idx_v)

    for ci in range(NCHUNK):
        off = ci * CHUNK
        pltpu.async_copy(
            wte_hbm.at[idx_v.at[pl.ds(off, CHUNK)]], tok_v, sem
        ).wait()
        pltpu.sync_copy(wpe_hbm.at[pl.ds(t_base + off, CHUNK)], pos_v)

        def add_row(r, _):
            for j in range(VPR):
                tok_v[r, pl.ds(j * L, L)] = (
                    tok_v[r, pl.ds(j * L, L)] + pos_v[r, pl.ds(j * L, L)]
                )
            return 0

        lax.fori_loop(0, CHUNK, add_row, 0)
        pltpu.sync_copy(tok_v, out_hbm.at[pl.ds(base + off, CHUNK)])


def kernel(x, wte, wpe):
    flat = _embed_lookup(x.reshape(-1).astype(jnp.int32), wte, wpe)
    return flat.reshape(BATCH, N_CTX, N_EMBED)
